# Initial kernel scaffold; baseline (speedup 1.0000x reference)
import jax
import jax.numpy as jnp
from jax.experimental import pallas as pl


def kernel(x, token_table, pos_table):
    raise NotImplementedError("write your pallas kernel here")


# speedup vs baseline: 2.3709x; 2.3709x over previous
import functools

import jax
import jax.numpy as jnp
from jax import lax
from jax.experimental import pallas as pl
from jax.experimental.pallas import tpu as pltpu
from jax.experimental.pallas import tpu_sc as plsc

BATCH = 4096
SEQ = 200
EMBED = 64
LANES = 16

NUM_CORES = 2
NUM_SUBCORES = 16
NW = NUM_CORES * NUM_SUBCORES
BATCH_PER_W = BATCH // NW
IDX_CHUNK = 100
CHUNKS_PER_W = BATCH_PER_W * SEQ // IDX_CHUNK


def _sc_body(x_hbm, tab_hbm, pos_hbm, out_hbm, idx_v, pos_v, buf_v, sem):
    wid = lax.axis_index("s") * NUM_CORES + lax.axis_index("c")

    pltpu.sync_copy(x_hbm.at[pl.ds(wid * CHUNKS_PER_W, CHUNKS_PER_W)], idx_v)
    pltpu.sync_copy(pos_hbm, pos_v)

    def batch_body(b, carry):
        cp0 = pltpu.make_async_copy(
            tab_hbm.at[idx_v.at[2 * b]], buf_v.at[pl.ds(0, IDX_CHUNK)], sem)
        cp1 = pltpu.make_async_copy(
            tab_hbm.at[idx_v.at[2 * b + 1]], buf_v.at[pl.ds(IDX_CHUNK, IDX_CHUNK)],
            sem)
        cp0.start()
        cp1.start()
        cp0.wait()
        cp1.wait()

        def s_body(s, c):
            for j in range(EMBED // LANES):
                sl = pl.ds(j * LANES, LANES)
                buf_v[s, sl] = buf_v[s, sl] + pos_v[s, sl]
            return c

        lax.fori_loop(0, SEQ, s_body, 0, unroll=2)

        pltpu.sync_copy(buf_v, out_hbm.at[wid * BATCH_PER_W + b])
        return carry

    lax.fori_loop(0, BATCH_PER_W, batch_body, 0)


@jax.jit
def kernel(x, token_table, pos_table):
    x_chunks = x.reshape(BATCH * SEQ // IDX_CHUNK, IDX_CHUNK)
    mesh = plsc.VectorSubcoreMesh(core_axis_name="c", subcore_axis_name="s")
    f = pl.kernel(
        _sc_body,
        out_type=jax.ShapeDtypeStruct((BATCH, SEQ, EMBED), jnp.float32),
        mesh=mesh,
        compiler_params=pltpu.CompilerParams(use_tc_tiling_on_sc=False),
        scratch_types=[
            pltpu.VMEM((CHUNKS_PER_W, IDX_CHUNK), jnp.int32),
            pltpu.VMEM((SEQ, EMBED), jnp.float32),
            pltpu.VMEM((SEQ, EMBED), jnp.float32),
            pltpu.SemaphoreType.DMA,
        ],
    )
    return f(x_chunks, token_table, pos_table)


# speedup vs baseline: 2.4546x; 1.0353x over previous
import functools

import jax
import jax.numpy as jnp
from jax import lax
from jax.experimental import pallas as pl
from jax.experimental.pallas import tpu as pltpu
from jax.experimental.pallas import tpu_sc as plsc

BATCH = 4096
SEQ = 200
EMBED = 64
LANES = 16

NUM_CORES = 2
NUM_SUBCORES = 16
NW = NUM_CORES * NUM_SUBCORES
BATCH_PER_W = BATCH // NW
IDX_CHUNK = 100
CHUNKS_PER_W = BATCH_PER_W * SEQ // IDX_CHUNK


def _sc_body(x_hbm, pidx_hbm, tab_hbm, pos_hbm, out_hbm, idx_v, pidx_v, buf_v,
             sem):
    wid = lax.axis_index("s") * NUM_CORES + lax.axis_index("c")

    pltpu.sync_copy(x_hbm.at[pl.ds(wid * CHUNKS_PER_W, CHUNKS_PER_W)], idx_v)
    pltpu.sync_copy(pidx_hbm, pidx_v)

    def batch_body(b, carry):
        cp0 = pltpu.async_copy(
            tab_hbm.at[idx_v.at[2 * b]], buf_v.at[pl.ds(0, IDX_CHUNK)], sem)
        cp1 = pltpu.async_copy(
            tab_hbm.at[idx_v.at[2 * b + 1]], buf_v.at[pl.ds(IDX_CHUNK, IDX_CHUNK)],
            sem)
        cp0.wait()
        cp1.wait()
        ca0 = pltpu.async_copy(
            pos_hbm.at[pidx_v.at[0]], buf_v.at[pl.ds(0, IDX_CHUNK)], sem,
            add=True)
        ca1 = pltpu.async_copy(
            pos_hbm.at[pidx_v.at[1]], buf_v.at[pl.ds(IDX_CHUNK, IDX_CHUNK)],
            sem, add=True)
        ca0.wait()
        ca1.wait()

        pltpu.sync_copy(buf_v, out_hbm.at[wid * BATCH_PER_W + b])
        return carry

    lax.fori_loop(0, BATCH_PER_W, batch_body, 0)


@jax.jit
def kernel(x, token_table, pos_table):
    x_chunks = x.reshape(BATCH * SEQ // IDX_CHUNK, IDX_CHUNK)
    pos_idx = jnp.arange(SEQ, dtype=jnp.int32).reshape(2, IDX_CHUNK)
    mesh = plsc.VectorSubcoreMesh(core_axis_name="c", subcore_axis_name="s")
    f = pl.kernel(
        _sc_body,
        out_type=jax.ShapeDtypeStruct((BATCH, SEQ, EMBED), jnp.float32),
        mesh=mesh,
        compiler_params=pltpu.CompilerParams(use_tc_tiling_on_sc=False),
        scratch_types=[
            pltpu.VMEM((CHUNKS_PER_W, IDX_CHUNK), jnp.int32),
            pltpu.VMEM((2, IDX_CHUNK), jnp.int32),
            pltpu.VMEM((SEQ, EMBED), jnp.float32),
            pltpu.SemaphoreType.DMA,
        ],
    )
    return f(x_chunks, pos_idx, token_table, pos_table)


# speedup vs baseline: 2.5093x; 1.0223x over previous
import jax
import jax.numpy as jnp
from jax import lax
from jax.experimental import pallas as pl
from jax.experimental.pallas import tpu as pltpu
from jax.experimental.pallas import tpu_sc as plsc

BATCH = 4096
SEQ = 200
EMBED = 64

NUM_CORES = 2
NUM_SUBCORES = 16
NW = NUM_CORES * NUM_SUBCORES
BATCH_PER_W = BATCH // NW
IDX_CHUNK = 100
CHUNKS_PER_W = BATCH_PER_W * SEQ // IDX_CHUNK
NB = 4


def _sc_body(x_hbm, pidx_hbm, tab_hbm, pos_hbm, out_hbm, idx_v, pidx_v, buf_v,
             sem_g, sem_p, sem_o):
    wid = lax.axis_index("s") * NUM_CORES + lax.axis_index("c")
    row0 = wid * BATCH_PER_W

    pltpu.sync_copy(x_hbm.at[pl.ds(wid * CHUNKS_PER_W, CHUNKS_PER_W)], idx_v)
    pltpu.sync_copy(pidx_hbm, pidx_v)

    halves = (pl.ds(0, IDX_CHUNK), pl.ds(IDX_CHUNK, IDX_CHUNK))

    def g_start(b, slot):
        for h in (0, 1):
            pltpu.async_copy(tab_hbm.at[idx_v.at[2 * b + h]],
                             buf_v.at[slot, halves[h]], sem_g.at[slot])

    def g_wait(slot):
        for h in (0, 1):
            pltpu.make_async_copy(tab_hbm.at[idx_v.at[h]],
                                  buf_v.at[slot, halves[h]],
                                  sem_g.at[slot]).wait()

    def p_start(slot):
        for h in (0, 1):
            pltpu.async_copy(pos_hbm.at[pidx_v.at[h]],
                             buf_v.at[slot, halves[h]], sem_p.at[slot],
                             add=True)

    def p_wait(slot):
        for h in (0, 1):
            pltpu.make_async_copy(pos_hbm.at[pidx_v.at[h]],
                                  buf_v.at[slot, halves[h]],
                                  sem_p.at[slot]).wait()

    def o_start(b, slot):
        pltpu.async_copy(buf_v.at[slot], out_hbm.at[row0 + b], sem_o.at[slot])

    def o_wait(b, slot):
        pltpu.make_async_copy(buf_v.at[slot], out_hbm.at[row0 + b],
                              sem_o.at[slot]).wait()

    g_start(0, 0)
    g_start(1, 1)
    g_wait(0)
    p_start(0)
    g_start(2, 2)
    g_wait(1)
    p_start(1)
    p_wait(0)
    o_start(0, 0)
    g_start(3, 3)
    g_wait(2)
    p_start(2)
    p_wait(1)
    o_start(1, 1)

    def body(b, carry):
        slot0 = lax.rem(b, NB)
        slot1 = lax.rem(b + 1, NB)
        slot2 = lax.rem(b + 2, NB)
        o_wait(b - 2, slot2)
        g_start(b + 2, slot2)
        g_wait(slot1)
        p_start(slot1)
        p_wait(slot0)
        o_start(b, slot0)
        return carry

    lax.fori_loop(2, BATCH_PER_W - 2, body, 0)

    b = BATCH_PER_W - 2
    o_wait(b - 2, (b + 2) % NB)
    g_wait((b + 1) % NB)
    p_start((b + 1) % NB)
    p_wait(b % NB)
    o_start(b, b % NB)
    b = BATCH_PER_W - 1
    o_wait(b - 2, (b + 2) % NB)
    p_wait(b % NB)
    o_start(b, b % NB)
    o_wait(BATCH_PER_W - 2, (BATCH_PER_W - 2) % NB)
    o_wait(BATCH_PER_W - 1, (BATCH_PER_W - 1) % NB)


@jax.jit
def kernel(x, token_table, pos_table):
    x_chunks = x.reshape(BATCH * SEQ // IDX_CHUNK, IDX_CHUNK)
    pos_idx = jnp.arange(SEQ, dtype=jnp.int32).reshape(2, IDX_CHUNK)
    mesh = plsc.VectorSubcoreMesh(core_axis_name="c", subcore_axis_name="s")
    f = pl.kernel(
        _sc_body,
        out_type=jax.ShapeDtypeStruct((BATCH, SEQ, EMBED), jnp.float32),
        mesh=mesh,
        compiler_params=pltpu.CompilerParams(use_tc_tiling_on_sc=False),
        scratch_types=[
            pltpu.VMEM((CHUNKS_PER_W, IDX_CHUNK), jnp.int32),
            pltpu.VMEM((2, IDX_CHUNK), jnp.int32),
            pltpu.VMEM((NB, SEQ, EMBED), jnp.float32),
            pltpu.SemaphoreType.DMA((NB,)),
            pltpu.SemaphoreType.DMA((NB,)),
            pltpu.SemaphoreType.DMA((NB,)),
        ],
    )
    return f(x_chunks, pos_idx, token_table, pos_table)
